# Initial kernel scaffold; baseline (speedup 1.0000x reference)
import jax
import jax.numpy as jnp
from jax.experimental import pallas as pl


def kernel(discount_spend, opt_sales, init_sales, opt_vol, brand_constraint, pack_constraint, price_segment_constraint, volume_variation_constraint, brand_gather_indices, pack_gather_indices, price_segment_gather_indices):
    raise NotImplementedError("write your pallas kernel here")


# speedup vs baseline: 1.5392x; 1.5392x over previous
import functools

import jax
import jax.numpy as jnp
from jax import lax
from jax.experimental import pallas as pl
from jax.experimental.pallas import tpu as pltpu
from jax.experimental.pallas import tpu_sc as plsc

_EPS = 1e-8
_ROI_LAMBDA = 10.0
_NEG_LAMBDA = 1000.0
_CONS_LAMBDA = 1000.0

_B, _N, _T = 8, 50000, 52
_NC, _NS, _L = 2, 16, 16
_NW = _NC * _NS
_W = 512

_BRAND_G, _BRAND_S = 500, 100
_PACK_G, _PACK_S = 1000, 50
_PS_G = 10
_PS_CHUNKS, _PS_CHUNK = 50, 100
_PS_UNITS = _PS_G * 4

_BRAND_WIN = 24
_PACK_WIN = 40
_PS_IDXWIN = 56
_BRAND_GPAD = 512
_PACK_GPAD = 1024
_PS_IDXPAD = 512


def _sc_body(tab, bgi, pgi, psi, bc, pc, psc, out,
             brow_v, krow_v, prow_v, bidx_v, kidx_v, pidx_v,
             bc_v, pc_v, psc_v, part_v, sem):
    cid = lax.axis_index("c")
    sid = lax.axis_index("s")
    wid = sid * _NC + cid

    lane = lax.iota(jnp.int32, _L)
    head4 = lane < 4
    zero16 = jnp.zeros((_L,), jnp.float32)

    part_v[...] = jnp.zeros((8, _L), jnp.float32)

    def srow(nch, buf):
        def body(s, accs):
            return tuple(accs[j] + buf[s, pl.ds(16 * j, 16)]
                         for j in range(nch))
        return body

    def block_relu(a0, a1, a2, a3, c0, c1, c2, c3):
        r0 = jnp.maximum(a0 - c0, 0.0)
        r1 = jnp.maximum(a1 - c1, 0.0)
        r2 = jnp.maximum(a2 - c2, 0.0)
        r3 = jnp.maximum(jnp.where(head4, a3 - c3, -1.0), 0.0)
        return r0 + r1 + r2 + r3

    def do_table(idx_hbm, g_total, s_cnt, idxbuf, cbuf, rowbuf):
        g_lo = (wid * g_total) // _NW
        g_hi = ((wid + 1) * g_total) // _NW
        g0 = 8 * (g_lo // 8)

        def body_g(g, carry):
            gl = g - g0
            pltpu.async_copy(tab.at[idxbuf.at[gl]], rowbuf, sem).wait()
            accs = lax.fori_loop(0, s_cnt, srow(32, rowbuf),
                                 (zero16,) * 32)
            c0 = cbuf[gl, pl.ds(0, 16)]
            c1 = cbuf[gl, pl.ds(16, 16)]
            c2 = cbuf[gl, pl.ds(32, 16)]
            c3 = cbuf[gl, pl.ds(48, 16)]
            tot = zero16
            for b in range(_B):
                tot = tot + block_relu(accs[4 * b], accs[4 * b + 1],
                                       accs[4 * b + 2], accs[4 * b + 3],
                                       c0, c1, c2, c3)
            part_v[0, :] = part_v[0, :] + tot
            return carry

        lax.fori_loop(g_lo, g_hi, body_g, 0)
        return g0

    bg0 = 8 * (((wid * _BRAND_G) // _NW) // 8)
    pltpu.sync_copy(bgi.at[pl.ds(bg0, _BRAND_WIN)], bidx_v)
    pltpu.sync_copy(bc.at[pl.ds(bg0, _BRAND_WIN)], bc_v)
    do_table(bgi, _BRAND_G, _BRAND_S, bidx_v, bc_v, brow_v)

    kg0 = 8 * (((wid * _PACK_G) // _NW) // 8)
    pltpu.sync_copy(pgi.at[pl.ds(kg0, _PACK_WIN)], kidx_v)
    pltpu.sync_copy(pc.at[pl.ds(kg0, _PACK_WIN)], pc_v)
    do_table(pgi, _PACK_G, _PACK_S, kidx_v, pc_v, krow_v)

    pltpu.sync_copy(psc.at[pl.ds(0, 16)], psc_v)
    u_lo = (wid * _PS_UNITS) // _NW
    u_hi = ((wid + 1) * _PS_UNITS) // _NW

    def body_u(u, carry):
        g = u // 4
        cb = u - g * 4
        r0 = g * _PS_CHUNKS
        ra = 8 * (r0 // 8)
        pltpu.sync_copy(psi.at[pl.ds(ra, _PS_IDXWIN)], pidx_v)
        roff = r0 - ra

        def body_k(k, accs):
            pltpu.async_copy(tab.at[pidx_v.at[roff + k],
                                    pl.ds(cb * 128, 128)],
                             prow_v, sem).wait()
            return lax.fori_loop(0, _PS_CHUNK, srow(8, prow_v), accs)

        accs = lax.fori_loop(0, _PS_CHUNKS, body_k, (zero16,) * 8)
        c0 = psc_v[g, pl.ds(0, 16)]
        c1 = psc_v[g, pl.ds(16, 16)]
        c2 = psc_v[g, pl.ds(32, 16)]
        c3 = psc_v[g, pl.ds(48, 16)]
        tot = block_relu(accs[0], accs[1], accs[2], accs[3], c0, c1, c2, c3)
        tot = tot + block_relu(accs[4], accs[5], accs[6], accs[7],
                               c0, c1, c2, c3)
        part_v[0, :] = part_v[0, :] + tot
        return carry

    lax.fori_loop(u_lo, u_hi, body_u, 0)

    pltpu.sync_copy(part_v, out.at[pl.ds(wid * 8, 8)])


@functools.lru_cache(maxsize=None)
def _get_sc_call():
    return pl.kernel(
        _sc_body,
        out_type=jax.ShapeDtypeStruct((_NW * 8, _L), jnp.float32),
        mesh=plsc.VectorSubcoreMesh(core_axis_name="c", subcore_axis_name="s",
                                    num_cores=_NC, num_subcores=_NS),
        scratch_types=[
            pltpu.VMEM((_BRAND_S, _W), jnp.float32),
            pltpu.VMEM((_PACK_S, _W), jnp.float32),
            pltpu.VMEM((_PS_CHUNK, 128), jnp.float32),
            pltpu.VMEM((_BRAND_WIN, _BRAND_S), jnp.int32),
            pltpu.VMEM((_PACK_WIN, _PACK_S), jnp.int32),
            pltpu.VMEM((_PS_IDXWIN, _PS_CHUNK), jnp.int32),
            pltpu.VMEM((_BRAND_WIN, 64), jnp.float32),
            pltpu.VMEM((_PACK_WIN, 64), jnp.float32),
            pltpu.VMEM((16, 64), jnp.float32),
            pltpu.VMEM((8, _L), jnp.float32),
            pltpu.SemaphoreType.DMA,
        ],
    )


_TC_BN = 1000


def _tc_body(vvc_ref, ds_ref, os_ref, is_ref, ov_ref, out_ref, tab_ref):
    @pl.when(pl.program_id(0) == 0)
    def _init():
        out_ref[...] = jnp.zeros_like(out_ref)

    ds = ds_ref[...]
    for b in range(_B):
        tab_ref[:, pl.ds(64 * b, _T)] = ds[b]

    s_ds = jnp.sum(ds, axis=(0, 1))
    s_neg = jnp.sum(jnp.maximum(-ds, 0.0), axis=(0, 1))
    s_os = jnp.sum(os_ref[...], axis=(0, 1))
    s_is = jnp.sum(is_ref[...], axis=(0, 1))
    ov = ov_ref[...]
    lo = vvc_ref[0]
    up = vvc_ref[1]
    s_vol = jnp.sum(jnp.maximum(ov - ov * up, 0.0)
                    + jnp.maximum(ov * lo - ov, 0.0), axis=(0, 1))
    zero = jnp.zeros_like(s_ds)
    out_ref[...] += jnp.stack([s_os, s_is, s_ds, s_neg, s_vol,
                               zero, zero, zero])


def _tc_call(vvc, ds, os_, is_, ov):
    grid = (_N // _TC_BN,)
    blk = pl.BlockSpec((_B, _TC_BN, _T), lambda i: (0, i, 0))
    return pl.pallas_call(
        _tc_body,
        grid=grid,
        in_specs=[pl.BlockSpec(memory_space=pltpu.SMEM), blk, blk, blk, blk],
        out_specs=[pl.BlockSpec((8, _T), lambda i: (0, 0)),
                   pl.BlockSpec((_TC_BN, _W), lambda i: (i, 0))],
        out_shape=[jax.ShapeDtypeStruct((8, _T), jnp.float32),
                   jax.ShapeDtypeStruct((_N, _W), jnp.float32)],
    )(vvc, ds, os_, is_, ov)


def _pad_rows_cols(x, rows, cols):
    return jnp.pad(x, ((0, rows - x.shape[0]), (0, cols - x.shape[1])))


def kernel(discount_spend, opt_sales, init_sales, opt_vol, brand_constraint,
           pack_constraint, price_segment_constraint,
           volume_variation_constraint, brand_gather_indices,
           pack_gather_indices, price_segment_gather_indices):
    psi2 = price_segment_gather_indices.reshape(_PS_G * _PS_CHUNKS, _PS_CHUNK)
    psi2 = jnp.pad(psi2, ((0, _PS_IDXPAD - psi2.shape[0]), (0, 0)))
    bgi = jnp.pad(brand_gather_indices, ((0, _BRAND_GPAD - _BRAND_G), (0, 0)))
    pgi = jnp.pad(pack_gather_indices, ((0, _PACK_GPAD - _PACK_G), (0, 0)))
    bc64 = _pad_rows_cols(brand_constraint, _BRAND_GPAD, 64)
    pc64 = _pad_rows_cols(pack_constraint, _PACK_GPAD, 64)
    psc64 = _pad_rows_cols(price_segment_constraint, 16, 64)

    tc_out, tab = _tc_call(volume_variation_constraint, discount_spend,
                           opt_sales, init_sales, opt_vol)

    sc_out = _get_sc_call()(tab, bgi, pgi, psi2, bc64, pc64, psc64)

    s_os = tc_out[0].sum()
    s_is = tc_out[1].sum()
    s_ds = tc_out[2].sum()
    s_neg = tc_out[3].sum()
    s_vol = tc_out[4].sum()
    cons = sc_out.sum()

    nr = s_os - s_is
    roi = nr / (s_ds + _EPS)
    return (-nr - _ROI_LAMBDA * roi + _NEG_LAMBDA * s_neg
            + _CONS_LAMBDA * cons + _CONS_LAMBDA * s_vol)


# speedup vs baseline: 2.0102x; 1.3060x over previous
import functools

import jax
import jax.numpy as jnp
from jax import lax
from jax.experimental import pallas as pl
from jax.experimental.pallas import tpu as pltpu
from jax.experimental.pallas import tpu_sc as plsc

_EPS = 1e-8
_ROI_LAMBDA = 10.0
_NEG_LAMBDA = 1000.0
_CONS_LAMBDA = 1000.0

_B, _N, _T = 8, 50000, 52
_NC, _NS, _L = 2, 16, 16
_NW = _NC * _NS
_W = 512

_BRAND_G, _BRAND_S = 500, 100
_PACK_G, _PACK_S = 1000, 50
_PS_G = 10
_PS_CHUNKS, _PS_CHUNK = 50, 100
_PS_UNITS = _PS_G * 4

_BRAND_WIN = 24
_PACK_WIN = 40
_PS_IDXWIN = 56
_BRAND_GPAD = 512
_PACK_GPAD = 1024
_PS_IDXPAD = 512


def _sc_body(tab, bgi, pgi, psi, bc, pc, psc, out,
             brow_v, krow_v, prow_v, bidx_v, kidx_v, pidx_v,
             bc_v, pc_v, psc_v, part_v, sem):
    cid = lax.axis_index("c")
    sid = lax.axis_index("s")
    wid = sid * _NC + cid

    lane = lax.iota(jnp.int32, _L)
    head4 = lane < 4
    zero16 = jnp.zeros((_L,), jnp.float32)

    part_v[...] = jnp.zeros((8, _L), jnp.float32)

    def srow(nch, buf):
        def body(s, accs):
            return tuple(accs[j] + buf[s, pl.ds(16 * j, 16)]
                         for j in range(nch))
        return body

    def block_relu(a0, a1, a2, a3, c0, c1, c2, c3):
        r0 = jnp.maximum(a0 - c0, 0.0)
        r1 = jnp.maximum(a1 - c1, 0.0)
        r2 = jnp.maximum(a2 - c2, 0.0)
        r3 = jnp.maximum(jnp.where(head4, a3 - c3, -1.0), 0.0)
        return r0 + r1 + r2 + r3

    def do_table(idx_hbm, g_total, s_cnt, idxbuf, cbuf, rowbuf):
        g_lo = (wid * g_total) // _NW
        g_hi = ((wid + 1) * g_total) // _NW
        g0 = 8 * (g_lo // 8)

        def body_g(g, carry):
            gl = g - g0
            pltpu.async_copy(tab.at[idxbuf.at[gl]], rowbuf, sem).wait()
            accs = lax.fori_loop(0, s_cnt, srow(32, rowbuf),
                                 (zero16,) * 32)
            c0 = cbuf[gl, pl.ds(0, 16)]
            c1 = cbuf[gl, pl.ds(16, 16)]
            c2 = cbuf[gl, pl.ds(32, 16)]
            c3 = cbuf[gl, pl.ds(48, 16)]
            tot = zero16
            for b in range(_B):
                tot = tot + block_relu(accs[4 * b], accs[4 * b + 1],
                                       accs[4 * b + 2], accs[4 * b + 3],
                                       c0, c1, c2, c3)
            part_v[0, :] = part_v[0, :] + tot
            return carry

        lax.fori_loop(g_lo, g_hi, body_g, 0)
        return g0

    bg0 = 8 * (((wid * _BRAND_G) // _NW) // 8)
    pltpu.sync_copy(bgi.at[pl.ds(bg0, _BRAND_WIN)], bidx_v)
    pltpu.sync_copy(bc.at[pl.ds(bg0, _BRAND_WIN)], bc_v)
    do_table(bgi, _BRAND_G, _BRAND_S, bidx_v, bc_v, brow_v)

    kg0 = 8 * (((wid * _PACK_G) // _NW) // 8)
    pltpu.sync_copy(pgi.at[pl.ds(kg0, _PACK_WIN)], kidx_v)
    pltpu.sync_copy(pc.at[pl.ds(kg0, _PACK_WIN)], pc_v)
    do_table(pgi, _PACK_G, _PACK_S, kidx_v, pc_v, krow_v)

    pltpu.sync_copy(psc.at[pl.ds(0, 16)], psc_v)
    u_lo = (wid * _PS_UNITS) // _NW
    u_hi = ((wid + 1) * _PS_UNITS) // _NW

    def body_u(u, carry):
        g = u // 4
        cb = u - g * 4
        r0 = g * _PS_CHUNKS
        ra = 8 * (r0 // 8)
        pltpu.sync_copy(psi.at[pl.ds(ra, _PS_IDXWIN)], pidx_v)
        roff = r0 - ra

        def body_k(k, accs):
            pltpu.async_copy(tab.at[pidx_v.at[roff + k],
                                    pl.ds(cb * 128, 128)],
                             prow_v, sem).wait()
            return lax.fori_loop(0, _PS_CHUNK, srow(8, prow_v), accs)

        accs = lax.fori_loop(0, _PS_CHUNKS, body_k, (zero16,) * 8)
        c0 = psc_v[g, pl.ds(0, 16)]
        c1 = psc_v[g, pl.ds(16, 16)]
        c2 = psc_v[g, pl.ds(32, 16)]
        c3 = psc_v[g, pl.ds(48, 16)]
        tot = block_relu(accs[0], accs[1], accs[2], accs[3], c0, c1, c2, c3)
        tot = tot + block_relu(accs[4], accs[5], accs[6], accs[7],
                               c0, c1, c2, c3)
        part_v[0, :] = part_v[0, :] + tot
        return carry

    lax.fori_loop(u_lo, u_hi, body_u, 0)

    pltpu.sync_copy(part_v, out.at[pl.ds(wid * 8, 8)])


@functools.lru_cache(maxsize=None)
def _get_sc_call():
    return pl.kernel(
        _sc_body,
        out_type=jax.ShapeDtypeStruct((_NW * 8, _L), jnp.float32),
        mesh=plsc.VectorSubcoreMesh(core_axis_name="c", subcore_axis_name="s",
                                    num_cores=_NC, num_subcores=_NS),
        scratch_types=[
            pltpu.VMEM((_BRAND_S, _W), jnp.float32),
            pltpu.VMEM((_PACK_S, _W), jnp.float32),
            pltpu.VMEM((_PS_CHUNK, 128), jnp.float32),
            pltpu.VMEM((_BRAND_WIN, _BRAND_S), jnp.int32),
            pltpu.VMEM((_PACK_WIN, _PACK_S), jnp.int32),
            pltpu.VMEM((_PS_IDXWIN, _PS_CHUNK), jnp.int32),
            pltpu.VMEM((_BRAND_WIN, 64), jnp.float32),
            pltpu.VMEM((_PACK_WIN, 64), jnp.float32),
            pltpu.VMEM((16, 64), jnp.float32),
            pltpu.VMEM((8, _L), jnp.float32),
            pltpu.SemaphoreType.DMA,
        ],
    )


_TC_BN = 2000


def _tc_ds_body(ds_ref, out_ref, tab_ref):
    @pl.when(pl.program_id(0) == 0)
    def _init():
        out_ref[...] = jnp.zeros_like(out_ref)

    ds = ds_ref[...]
    for b in range(_B):
        tab_ref[:, pl.ds(64 * b, _T)] = ds[b]

    s_ds = jnp.sum(ds, axis=(0, 1))
    s_neg = jnp.sum(jnp.maximum(-ds, 0.0), axis=(0, 1))
    zero = jnp.zeros_like(s_ds)
    out_ref[...] += jnp.stack([s_ds, s_neg, zero, zero,
                               zero, zero, zero, zero])


def _tc_ds_call(ds):
    grid = (_N // _TC_BN,)
    blk = pl.BlockSpec((_B, _TC_BN, _T), lambda i: (0, i, 0))
    return pl.pallas_call(
        _tc_ds_body,
        grid=grid,
        in_specs=[blk],
        out_specs=[pl.BlockSpec((8, _T), lambda i: (0, 0)),
                   pl.BlockSpec((_TC_BN, _W), lambda i: (i, 0))],
        out_shape=[jax.ShapeDtypeStruct((8, _T), jnp.float32),
                   jax.ShapeDtypeStruct((_N, _W), jnp.float32)],
    )(ds)


def _tc_rest_body(vvc_ref, os_ref, is_ref, ov_ref, out_ref):
    @pl.when(pl.program_id(0) == 0)
    def _init():
        out_ref[...] = jnp.zeros_like(out_ref)

    s_os = jnp.sum(os_ref[...], axis=(0, 1))
    s_is = jnp.sum(is_ref[...], axis=(0, 1))
    ov = ov_ref[...]
    lo = vvc_ref[0]
    up = vvc_ref[1]
    s_vol = jnp.sum(jnp.maximum(ov - ov * up, 0.0)
                    + jnp.maximum(ov * lo - ov, 0.0), axis=(0, 1))
    zero = jnp.zeros_like(s_os)
    out_ref[...] += jnp.stack([s_os, s_is, s_vol, zero,
                               zero, zero, zero, zero])


def _tc_rest_call(vvc, os_, is_, ov):
    grid = (_N // _TC_BN,)
    blk = pl.BlockSpec((_B, _TC_BN, _T), lambda i: (0, i, 0))
    return pl.pallas_call(
        _tc_rest_body,
        grid=grid,
        in_specs=[pl.BlockSpec(memory_space=pltpu.SMEM), blk, blk, blk],
        out_specs=pl.BlockSpec((8, _T), lambda i: (0, 0)),
        out_shape=jax.ShapeDtypeStruct((8, _T), jnp.float32),
    )(vvc, os_, is_, ov)


def _pad_rows_cols(x, rows, cols):
    return jnp.pad(x, ((0, rows - x.shape[0]), (0, cols - x.shape[1])))


def kernel(discount_spend, opt_sales, init_sales, opt_vol, brand_constraint,
           pack_constraint, price_segment_constraint,
           volume_variation_constraint, brand_gather_indices,
           pack_gather_indices, price_segment_gather_indices):
    psi2 = price_segment_gather_indices.reshape(_PS_G * _PS_CHUNKS, _PS_CHUNK)
    psi2 = jnp.pad(psi2, ((0, _PS_IDXPAD - psi2.shape[0]), (0, 0)))
    bgi = jnp.pad(brand_gather_indices, ((0, _BRAND_GPAD - _BRAND_G), (0, 0)))
    pgi = jnp.pad(pack_gather_indices, ((0, _PACK_GPAD - _PACK_G), (0, 0)))
    bc64 = _pad_rows_cols(brand_constraint, _BRAND_GPAD, 64)
    pc64 = _pad_rows_cols(pack_constraint, _PACK_GPAD, 64)
    psc64 = _pad_rows_cols(price_segment_constraint, 16, 64)

    ds_out, tab = _tc_ds_call(discount_spend)

    sc_out = _get_sc_call()(tab, bgi, pgi, psi2, bc64, pc64, psc64)

    rest_out = _tc_rest_call(volume_variation_constraint, opt_sales,
                             init_sales, opt_vol)

    s_ds = ds_out[0].sum()
    s_neg = ds_out[1].sum()
    s_os = rest_out[0].sum()
    s_is = rest_out[1].sum()
    s_vol = rest_out[2].sum()
    cons = sc_out.sum()

    nr = s_os - s_is
    roi = nr / (s_ds + _EPS)
    return (-nr - _ROI_LAMBDA * roi + _NEG_LAMBDA * s_neg
            + _CONS_LAMBDA * cons + _CONS_LAMBDA * s_vol)
